# baseline (speedup 1.0000x reference)
import functools

import jax
import jax.numpy as jnp
from jax import lax
from jax.experimental import pallas as pl
from jax.experimental.pallas import tpu as pltpu
from jax.experimental.pallas import tpu_sc as plsc

N = 10000
E = 320000
NFEATS = 128
NHIDS = 128
NHEADS = 8
DHEAD = 16
ALPHA = 0.2
CC = 0.5

NB_ROWS = 2000
EB = 125
NC, NS = 2, 16
NW = NC * NS
EPW = E // NW
NBLK = EPW // EB
RPT = N // NS


def _prep_body(h_ref, w_ref, b_ref, wp_ref, bp_ref, asrc_ref, adst_ref,
               ba_ref, t_ref, whbf_ref):
    wh = lax.dot_general(h_ref[...], w_ref[...], (((1,), (1,)), ((), ())),
                         preferred_element_type=jnp.float32) + b_ref[...]
    ts = lax.dot_general(wh, asrc_ref[...], (((1,), (0,)), ((), ())),
                         preferred_element_type=jnp.float32)
    td = lax.dot_general(wh, adst_ref[...], (((1,), (0,)), ((), ())),
                         preferred_element_type=jnp.float32) + ba_ref[...]
    t_ref[...] = jnp.concatenate([ts, td], axis=1)
    whp = lax.dot_general(h_ref[...], wp_ref[...], (((1,), (1,)), ((), ())),
                          preferred_element_type=jnp.float32) + bp_ref[...]
    bits = lax.bitcast_convert_type(whp, jnp.int32)
    rnd = bits + 32767 + jnp.bitwise_and(
        lax.shift_right_logical(bits, 16), 1)
    lo = lax.shift_right_logical(rnd[:, :64], 16)
    hi = jnp.bitwise_and(rnd[:, 64:], jnp.int32(-65536))
    whbf_ref[...] = jnp.bitwise_or(lo, hi)


def _prep(h, w2, bflat, w2p, bp, a_src, a_dst, ba_row):
    return pl.pallas_call(
        _prep_body,
        grid=(N // NB_ROWS,),
        in_specs=[
            pl.BlockSpec((NB_ROWS, NFEATS), lambda i: (i, 0)),
            pl.BlockSpec((NHIDS, NFEATS), lambda i: (0, 0)),
            pl.BlockSpec((1, NHIDS), lambda i: (0, 0)),
            pl.BlockSpec((NHIDS, NFEATS), lambda i: (0, 0)),
            pl.BlockSpec((1, NHIDS), lambda i: (0, 0)),
            pl.BlockSpec((NHIDS, NHEADS), lambda i: (0, 0)),
            pl.BlockSpec((NHIDS, NHEADS), lambda i: (0, 0)),
            pl.BlockSpec((1, NHEADS), lambda i: (0, 0)),
        ],
        out_specs=[
            pl.BlockSpec((NB_ROWS, 2 * NHEADS), lambda i: (i, 0)),
            pl.BlockSpec((NB_ROWS, NHIDS // 2), lambda i: (i, 0)),
        ],
        out_shape=[
            jax.ShapeDtypeStruct((N, 2 * NHEADS), jnp.float32),
            jax.ShapeDtypeStruct((N, NHIDS // 2), jnp.int32),
        ],
    )(h, w2, bflat, w2p, bp, a_src, a_dst, ba_row)


def _lane_bcast(vec, lane):
    idx = jnp.full((16, 1), lane, jnp.int32)
    return lax.gather(
        vec, idx,
        lax.GatherDimensionNumbers(offset_dims=(), collapsed_slice_dims=(0,),
                                   start_index_map=(0,)),
        (1,), mode=lax.GatherScatterMode.PROMISE_IN_BOUNDS)


def _sc_body(wh_t, wh_p, wh_p2, tt, tp, tp2, tcur, s0, d0, s1, d1, s2, d2,
             acc_out, den_out,
             ixs, ixd, sA, sB, exb, rowsbf, rowsf,
             acc_sh, den_sh, sga, sgb, sgr):
    c = lax.axis_index("c")
    s = lax.axis_index("s")
    wid = c * NS + s
    zero16 = jnp.zeros((16,), jnp.float32)

    r0 = s * RPT
    layers = (
        (wh_t, tt, tt, s0, d0),
        (wh_p, tp, tcur, s1, d1),
        (wh_p2, tp2, tcur, s2, d2),
    )
    for l, (wh, ts_tab, td_tab, se, de) in enumerate(layers):
        def _zr(i, u):
            rowsf[i // 8, pl.ds((i % 8) * 16, 16)] = zero16
            return u
        lax.fori_loop(0, 25 * 8, _zr, 0)

        def _zd(i, u):
            exb[i, :] = zero16
            return u
        lax.fori_loop(0, EB, _zd, 0)
        for k in range(25):
            pltpu.sync_copy(rowsf.at[pl.ds(0, 25)],
                            acc_sh.at[pl.ds(r0 + k * 25, 25)])
        for k in range(5):
            pltpu.sync_copy(exb, den_sh.at[pl.ds(r0 + k * EB, EB)])
        plsc.subcore_barrier()

        row_base = wid * NBLK

        def _blk(bi, u):
            pltpu.sync_copy(se.at[pl.ds(row_base + bi, 1)], ixs)
            pltpu.sync_copy(de.at[pl.ds(row_base + bi, 1)], ixd)
            cpa = pltpu.async_copy(ts_tab.at[ixs.at[0]], sA, sga)
            cpb = pltpu.async_copy(td_tab.at[ixd.at[0]], sB, sgb)
            cpr = pltpu.async_copy(wh.at[ixs.at[0]], rowsbf, sgr)
            cpa.wait()
            cpb.wait()

            def _score(b, v):
                e = sA[b, :] + lax.rev(sB[b, :], dimensions=(0,))
                e = jnp.where(e >= 0, e, ALPHA * e)
                exb[b, :] = jnp.exp(e)
                return v
            lax.fori_loop(0, EB, _score, 0)
            pltpu.sync_copy(exb, den_sh.at[ixd.at[0]], add=True)
            cpr.wait()

            def _mul(b, v):
                e8 = exb[b, :]
                for g in range(4):
                    w = rowsbf[b, pl.ds(16 * g, 16)]
                    u0 = lax.bitcast_convert_type(
                        lax.shift_left(w, 16), jnp.float32)
                    u1 = lax.bitcast_convert_type(
                        jnp.bitwise_and(w, jnp.int32(-65536)), jnp.float32)
                    rowsf[b, pl.ds(32 * g, 16)] = (
                        u0 * _lane_bcast(e8, 2 * g))
                    rowsf[b, pl.ds(32 * g + 16, 16)] = (
                        u1 * _lane_bcast(e8, 2 * g + 1))
                return v
            lax.fori_loop(0, EB, _mul, 0)
            pltpu.sync_copy(rowsf, acc_sh.at[ixd.at[0]], add=True)
            return u
        lax.fori_loop(0, NBLK, _blk, 0)
        plsc.subcore_barrier()

        r0c = s * 624
        pltpu.sync_copy(acc_sh.at[pl.ds(r0c, 624)],
                        acc_out.at[l, c, pl.ds(r0c, 624)])
        pltpu.sync_copy(den_sh.at[pl.ds(r0c, 624)],
                        den_out.at[l, c, pl.ds(r0c, 624)])

        @pl.when(s == 0)
        def _tail():
            pltpu.sync_copy(acc_sh.at[pl.ds(9984, 16)],
                            acc_out.at[l, c, pl.ds(9984, 16)])
            pltpu.sync_copy(den_sh.at[pl.ds(9984, 16)],
                            den_out.at[l, c, pl.ds(9984, 16)])
        plsc.subcore_barrier()


def _sc_edges(wh_t, wh_p, wh_p2, tt, tp, tp2, tcur, s0, d0, s1, d1, s2, d2):
    mesh = plsc.VectorSubcoreMesh(core_axis_name="c", subcore_axis_name="s")
    fn = pl.kernel(
        _sc_body,
        mesh=mesh,
        out_type=[
            jax.ShapeDtypeStruct((3, NC, N, NHIDS), jnp.float32),
            jax.ShapeDtypeStruct((3, NC, N, 16), jnp.float32),
        ],
        scratch_types=(
            [pltpu.VMEM((1, EB), jnp.int32)] * 2
            + [pltpu.VMEM((EB, 16), jnp.float32)] * 3
            + [pltpu.VMEM((EB, NHIDS // 2), jnp.int32),
               pltpu.VMEM((EB, NHIDS), jnp.float32),
               pltpu.VMEM_SHARED((N, NHIDS), jnp.float32),
               pltpu.VMEM_SHARED((N, 16), jnp.float32)]
            + [pltpu.SemaphoreType.DMA] * 3
        ),
        compiler_params=pltpu.CompilerParams(use_tc_tiling_on_sc=False),
    )
    return fn(wh_t, wh_p, wh_p2, tt, tp, tp2, tcur, s0, d0, s1, d1, s2, d2)


def _finish_body(acc_ref, den_ref, wih_ref, whh_ref, bih_ref, bhh_ref,
                 coef_ref, out_ref):
    outs = []
    for l in range(3):
        an = acc_ref[l, 0] + acc_ref[l, 1]
        dn = den_ref[l, 0, :, :NHEADS] + den_ref[l, 1, :, :NHEADS]
        inv = jnp.where(dn > 0, 1.0 / dn, 0.0)
        parts = [an[:, h * DHEAD:(h + 1) * DHEAD] * inv[:, h:h + 1]
                 for h in range(NHEADS)]
        outs.append(jnp.concatenate(parts, axis=1))
    x, hc, hs = outs
    ccf = coef_ref[0, 0]
    csf = coef_ref[0, 1]
    g = coef_ref[0, 2]
    h = ccf * hc + csf * hs
    gi = lax.dot_general(x, wih_ref[...], (((1,), (1,)), ((), ())),
                         preferred_element_type=jnp.float32) + bih_ref[...]
    gh = lax.dot_general(h, whh_ref[...], (((1,), (1,)), ((), ())),
                         preferred_element_type=jnp.float32) + bhh_ref[...]
    r = jax.nn.sigmoid(gi[:, :NHIDS] + gh[:, :NHIDS])
    z = jax.nn.sigmoid(gi[:, NHIDS:2 * NHIDS] + gh[:, NHIDS:2 * NHIDS])
    nn = jnp.tanh(gi[:, 2 * NHIDS:] + r * gh[:, 2 * NHIDS:])
    out = (1.0 - z) * nn + z * h
    out_ref[...] = g * out + (1.0 - g) * x


def _finish(acc, den, wih, whh, bih, bhh, coef):
    return pl.pallas_call(
        _finish_body,
        grid=(N // NB_ROWS,),
        in_specs=[
            pl.BlockSpec((3, NC, NB_ROWS, NHIDS), lambda i: (0, 0, i, 0)),
            pl.BlockSpec((3, NC, NB_ROWS, 16), lambda i: (0, 0, i, 0)),
            pl.BlockSpec((3 * NHIDS, NHIDS), lambda i: (0, 0)),
            pl.BlockSpec((3 * NHIDS, NHIDS), lambda i: (0, 0)),
            pl.BlockSpec((1, 3 * NHIDS), lambda i: (0, 0)),
            pl.BlockSpec((1, 3 * NHIDS), lambda i: (0, 0)),
            pl.BlockSpec(memory_space=pltpu.SMEM),
        ],
        out_specs=pl.BlockSpec((NB_ROWS, NHIDS), lambda i: (i, 0)),
        out_shape=jax.ShapeDtypeStruct((N, NHIDS), jnp.float32),
    )(acc, den, wih, whh, bih, bhh, coef)


def kernel(h_t, hp_prev, hp_prev2, hp_cur, edge_index_intra,
           edge_index_counter, edge_index_support, W_gat, b_gat, a_gat,
           ba_gat, W_x, b_x, a_x, ba_x, weight_ih, weight_hh, bias_ih,
           bias_hh, t):
    f32 = jnp.float32
    Wg = W_gat.reshape(NHIDS, NFEATS)
    Wx = W_x.reshape(NHIDS, NFEATS)
    bg = b_gat.reshape(1, NHIDS)
    bx = b_x.reshape(1, NHIDS)
    eye = jnp.eye(NHEADS, dtype=f32)

    k = jnp.arange(NHIDS // 2)
    oca = 32 * (k // DHEAD) + k % DHEAD
    oc = jnp.concatenate([oca, oca + DHEAD])
    Wgp = Wg[oc]
    Wxp = Wx[oc]
    bgp = bg[:, oc]
    bxp = bx[:, oc]

    def amats(a):
        a_src = (a[:, :DHEAD, None] * eye[:, None, :]).reshape(NHIDS, NHEADS)
        a_dst = (a[:, DHEAD:, None] * eye[:, None, :]).reshape(NHIDS, NHEADS)
        return a_src, a_dst[:, ::-1]

    asg, adg = amats(a_gat)
    asx, adx = amats(a_x)
    bag = ba_gat[::-1].reshape(1, NHEADS)
    bax = ba_x[::-1].reshape(1, NHEADS)

    t_t, whb_t = _prep(h_t, Wg, bg, Wgp, bgp, asg, adg, bag)
    t_p, whb_p = _prep(hp_prev, Wx, bx, Wxp, bxp, asx, adx, bax)
    t_p2, whb_p2 = _prep(hp_prev2, Wx, bx, Wxp, bxp, asx, adx, bax)
    t_c, _ = _prep(hp_cur, Wx, bx, Wxp, bxp, asx, adx, bax)

    s0 = edge_index_intra[0].reshape(E // EB, EB)
    d0 = edge_index_intra[1].reshape(E // EB, EB)
    s1 = edge_index_counter[0].reshape(E // EB, EB)
    d1 = edge_index_counter[1].reshape(E // EB, EB)
    s2 = edge_index_support[0].reshape(E // EB, EB)
    d2 = edge_index_support[1].reshape(E // EB, EB)

    acc, den = _sc_edges(whb_t, whb_p, whb_p2, t_t, t_p, t_p2, t_c,
                         s0, d0, s1, d1, s2, d2)

    tv = jnp.asarray(t)
    ccf = jnp.where(tv > 1, CC, 1.0).astype(f32)
    csf = jnp.where(tv > 1, 1.0 - CC, 0.0).astype(f32)
    g = jnp.where(tv > 0, 1.0, 0.0).astype(f32)
    coef = jnp.stack([ccf, csf, g]).reshape(1, 3)

    return _finish(acc, den, weight_ih, weight_hh,
                   bias_ih.reshape(1, -1), bias_hh.reshape(1, -1), coef)


# speedup vs baseline: 1.2975x; 1.2975x over previous
import functools

import jax
import jax.numpy as jnp
from jax import lax
from jax.experimental import pallas as pl
from jax.experimental.pallas import tpu as pltpu
from jax.experimental.pallas import tpu_sc as plsc

N = 10000
E = 320000
NFEATS = 128
NHIDS = 128
NHEADS = 8
DHEAD = 16
ALPHA = 0.2
CC = 0.5

NB_ROWS = 2000
EB = 100
WROW = NHIDS + 16
NC, NS = 2, 16
NW = NC * NS
EPW = E // NW
NBLK = EPW // EB
RPT = N // NS


def _prep_body(h_ref, w_ref, b_ref, asrc_ref, adst_ref, ba_ref,
               t_ref, whp_ref):
    wh = lax.dot_general(h_ref[...], w_ref[...], (((1,), (1,)), ((), ())),
                         preferred_element_type=jnp.float32) + b_ref[...]
    ts = lax.dot_general(wh, asrc_ref[...], (((1,), (0,)), ((), ())),
                         preferred_element_type=jnp.float32)
    td = lax.dot_general(wh, adst_ref[...], (((1,), (0,)), ((), ())),
                         preferred_element_type=jnp.float32) + ba_ref[...]
    t_ref[...] = jnp.concatenate([ts, td], axis=1)
    whp_ref[...] = jnp.concatenate([wh, ts, td], axis=1)


def _prep(h, w2, bflat, a_src, a_dst, ba_row):
    return pl.pallas_call(
        _prep_body,
        grid=(N // NB_ROWS,),
        in_specs=[
            pl.BlockSpec((NB_ROWS, NFEATS), lambda i: (i, 0)),
            pl.BlockSpec((NHIDS, NFEATS), lambda i: (0, 0)),
            pl.BlockSpec((1, NHIDS), lambda i: (0, 0)),
            pl.BlockSpec((NHIDS, NHEADS), lambda i: (0, 0)),
            pl.BlockSpec((NHIDS, NHEADS), lambda i: (0, 0)),
            pl.BlockSpec((1, NHEADS), lambda i: (0, 0)),
        ],
        out_specs=[
            pl.BlockSpec((NB_ROWS, 2 * NHEADS), lambda i: (i, 0)),
            pl.BlockSpec((NB_ROWS, WROW), lambda i: (i, 0)),
        ],
        out_shape=[
            jax.ShapeDtypeStruct((N, 2 * NHEADS), jnp.float32),
            jax.ShapeDtypeStruct((N, WROW), jnp.float32),
        ],
    )(h, w2, bflat, a_src, a_dst, ba_row)


def _lane_bcast(vec, lane):
    idx = jnp.full((16, 1), lane, jnp.int32)
    return lax.gather(
        vec, idx,
        lax.GatherDimensionNumbers(offset_dims=(), collapsed_slice_dims=(0,),
                                   start_index_map=(0,)),
        (1,), mode=lax.GatherScatterMode.PROMISE_IN_BOUNDS)


def _sc_body(wh_t, wh_p, wh_p2, tt, tp, tp2, tcur, s0, d0, s1, d1, s2, d2,
             acc_out,
             ixsA, ixgA, ixdA, ixsB, ixgB, ixdB, sbA, sbB, rwA, rwB,
             acc_sh,
             sgrA, sgbA, sxgA, sxsA, ssaA,
             sgrB, sgbB, sxgB, sxsB, ssaB):
    c = lax.axis_index("c")
    s = lax.axis_index("s")
    wid = c * NS + s
    zero16 = jnp.zeros((16,), jnp.float32)

    A = (ixsA, ixgA, ixdA, sbA, rwA, sgrA, sgbA, sxgA, sxsA, ssaA)
    B = (ixsB, ixgB, ixdB, sbB, rwB, sgrB, sgbB, sxgB, sxsB, ssaB)

    r0 = s * RPT
    layers = (
        (wh_t, tt, s0, d0),
        (wh_p, tcur, s1, d1),
        (wh_p2, tcur, s2, d2),
    )
    for l, (wh, td_tab, se, de) in enumerate(layers):
        def _zr(i, u):
            rwA[i // 9, pl.ds((i % 9) * 16, 16)] = zero16
            return u
        lax.fori_loop(0, 25 * 9, _zr, 0)
        for k in range(25):
            pltpu.sync_copy(rwA.at[pl.ds(0, 25)],
                            acc_sh.at[pl.ds(r0 + k * 25, 25)])
        plsc.subcore_barrier()

        row_base = wid * NBLK

        def _issue_ixg(bi, S):
            pltpu.async_copy(se.at[pl.ds(row_base + bi, 1)], S[0], S[7])
            pltpu.async_copy(de.at[pl.ds(row_base + bi, 1)], S[1], S[7])

        def _drain_ixg(S):
            pltpu.make_async_copy(se.at[pl.ds(row_base, 1)], S[0], S[7]).wait()
            pltpu.make_async_copy(de.at[pl.ds(row_base, 1)], S[1], S[7]).wait()

        def _issue_ixs(bi, S):
            pltpu.async_copy(de.at[pl.ds(row_base + bi, 1)], S[2], S[8])

        def _drain_ixs(S):
            pltpu.make_async_copy(de.at[pl.ds(row_base, 1)], S[2], S[8]).wait()

        def _issue_gathers(S):
            pltpu.async_copy(wh.at[S[0].at[0]], S[4], S[5])
            pltpu.async_copy(td_tab.at[S[1].at[0]], S[3], S[6])

        def _drain_gathers(S):
            pltpu.make_async_copy(wh.at[S[0].at[0]], S[4], S[5]).wait()
            pltpu.make_async_copy(td_tab.at[S[1].at[0]], S[3], S[6]).wait()

        def _issue_scatter(S):
            pltpu.async_copy(S[4], acc_sh.at[S[2].at[0]], S[9], add=True)

        def _drain_scatter(S):
            pltpu.make_async_copy(S[4], acc_sh.at[S[2].at[0]], S[9]).wait()

        def _compute(S):
            sB, rows = S[3], S[4]

            def _score(b, v):
                e = rows[b, pl.ds(NHIDS, 16)] + lax.rev(sB[b, :],
                                                        dimensions=(0,))
                e = jnp.where(e >= 0, e, ALPHA * e)
                rows[b, pl.ds(NHIDS, 16)] = jnp.exp(e)
                return v
            lax.fori_loop(0, EB, _score, 0)

            def _mul(b, v):
                e8 = rows[b, pl.ds(NHIDS, 16)]
                for h in range(NHEADS):
                    rows[b, pl.ds(h * DHEAD, DHEAD)] = (
                        rows[b, pl.ds(h * DHEAD, DHEAD)] * _lane_bcast(e8, h))
                return v
            lax.fori_loop(0, EB, _mul, 0)

        def _half(b, S, T):
            _drain_gathers(S)
            _drain_ixs(S)
            _compute(S)
            _issue_scatter(S)
            _issue_ixg(jnp.minimum(b + 2, NBLK - 1), S)

            @pl.when(b >= 1)
            def _():
                _drain_scatter(T)
            _issue_ixs(jnp.minimum(b + 1, NBLK - 1), T)
            _drain_ixg(T)
            _issue_gathers(T)

        _issue_ixg(0, A)
        _issue_ixs(0, A)
        _drain_ixg(A)
        _issue_gathers(A)
        _issue_ixg(1, B)

        def _pair(i, u):
            _half(2 * i, A, B)
            _half(2 * i + 1, B, A)
            return u
        lax.fori_loop(0, NBLK // 2, _pair, 0)

        _drain_gathers(A)
        _drain_ixg(B)
        _drain_ixs(A)
        _drain_scatter(B)
        plsc.subcore_barrier()

        r0c = s * 624
        pltpu.sync_copy(acc_sh.at[pl.ds(r0c, 624)],
                        acc_out.at[l, c, pl.ds(r0c, 624)])

        @pl.when(s == 0)
        def _tail():
            pltpu.sync_copy(acc_sh.at[pl.ds(9984, 16)],
                            acc_out.at[l, c, pl.ds(9984, 16)])
        plsc.subcore_barrier()


def _sc_edges(wh_t, wh_p, wh_p2, tt, tp, tp2, tcur, s0, d0, s1, d1, s2, d2):
    mesh = plsc.VectorSubcoreMesh(core_axis_name="c", subcore_axis_name="s")
    fn = pl.kernel(
        _sc_body,
        mesh=mesh,
        out_type=[
            jax.ShapeDtypeStruct((3, NC, N, WROW), jnp.float32),
        ],
        scratch_types=(
            [pltpu.VMEM((1, EB), jnp.int32)] * 6
            + [pltpu.VMEM((EB, 16), jnp.float32)] * 2
            + [pltpu.VMEM((EB, WROW), jnp.float32)] * 2
            + [pltpu.VMEM_SHARED((N, WROW), jnp.float32)]
            + [pltpu.SemaphoreType.DMA] * 10
        ),
        compiler_params=pltpu.CompilerParams(use_tc_tiling_on_sc=False),
    )
    return fn(wh_t, wh_p, wh_p2, tt, tp, tp2, tcur,
              s0, d0, s1, d1, s2, d2)[0]


def _finish_body(acc_ref, wih_ref, whh_ref, bih_ref, bhh_ref,
                 coef_ref, out_ref):
    outs = []
    for l in range(3):
        an = acc_ref[l, 0, :, :NHIDS] + acc_ref[l, 1, :, :NHIDS]
        dn = (acc_ref[l, 0, :, NHIDS:NHIDS + NHEADS]
              + acc_ref[l, 1, :, NHIDS:NHIDS + NHEADS])
        inv = jnp.where(dn > 0, 1.0 / dn, 0.0)
        parts = [an[:, h * DHEAD:(h + 1) * DHEAD] * inv[:, h:h + 1]
                 for h in range(NHEADS)]
        outs.append(jnp.concatenate(parts, axis=1))
    x, hc, hs = outs
    ccf = coef_ref[0, 0]
    csf = coef_ref[0, 1]
    g = coef_ref[0, 2]
    h = ccf * hc + csf * hs
    gi = lax.dot_general(x, wih_ref[...], (((1,), (1,)), ((), ())),
                         preferred_element_type=jnp.float32) + bih_ref[...]
    gh = lax.dot_general(h, whh_ref[...], (((1,), (1,)), ((), ())),
                         preferred_element_type=jnp.float32) + bhh_ref[...]
    r = jax.nn.sigmoid(gi[:, :NHIDS] + gh[:, :NHIDS])
    z = jax.nn.sigmoid(gi[:, NHIDS:2 * NHIDS] + gh[:, NHIDS:2 * NHIDS])
    nn = jnp.tanh(gi[:, 2 * NHIDS:] + r * gh[:, 2 * NHIDS:])
    out = (1.0 - z) * nn + z * h
    out_ref[...] = g * out + (1.0 - g) * x


def _finish(acc, wih, whh, bih, bhh, coef):
    return pl.pallas_call(
        _finish_body,
        grid=(N // NB_ROWS,),
        in_specs=[
            pl.BlockSpec((3, NC, NB_ROWS, WROW), lambda i: (0, 0, i, 0)),
            pl.BlockSpec((3 * NHIDS, NHIDS), lambda i: (0, 0)),
            pl.BlockSpec((3 * NHIDS, NHIDS), lambda i: (0, 0)),
            pl.BlockSpec((1, 3 * NHIDS), lambda i: (0, 0)),
            pl.BlockSpec((1, 3 * NHIDS), lambda i: (0, 0)),
            pl.BlockSpec(memory_space=pltpu.SMEM),
        ],
        out_specs=pl.BlockSpec((NB_ROWS, NHIDS), lambda i: (i, 0)),
        out_shape=jax.ShapeDtypeStruct((N, NHIDS), jnp.float32),
    )(acc, wih, whh, bih, bhh, coef)


def kernel(h_t, hp_prev, hp_prev2, hp_cur, edge_index_intra,
           edge_index_counter, edge_index_support, W_gat, b_gat, a_gat,
           ba_gat, W_x, b_x, a_x, ba_x, weight_ih, weight_hh, bias_ih,
           bias_hh, t):
    f32 = jnp.float32
    Wg = W_gat.reshape(NHIDS, NFEATS)
    Wx = W_x.reshape(NHIDS, NFEATS)
    bg = b_gat.reshape(1, NHIDS)
    bx = b_x.reshape(1, NHIDS)
    eye = jnp.eye(NHEADS, dtype=f32)

    def amats(a):
        a_src = (a[:, :DHEAD, None] * eye[:, None, :]).reshape(NHIDS, NHEADS)
        a_dst = (a[:, DHEAD:, None] * eye[:, None, :]).reshape(NHIDS, NHEADS)
        return a_src, a_dst[:, ::-1]

    asg, adg = amats(a_gat)
    asx, adx = amats(a_x)
    bag = ba_gat[::-1].reshape(1, NHEADS)
    bax = ba_x[::-1].reshape(1, NHEADS)

    t_t, whp_t = _prep(h_t, Wg, bg, asg, adg, bag)
    t_p, whp_p = _prep(hp_prev, Wx, bx, asx, adx, bax)
    t_p2, whp_p2 = _prep(hp_prev2, Wx, bx, asx, adx, bax)
    t_c, _ = _prep(hp_cur, Wx, bx, asx, adx, bax)

    s0 = edge_index_intra[0].reshape(E // EB, EB)
    d0 = edge_index_intra[1].reshape(E // EB, EB)
    s1 = edge_index_counter[0].reshape(E // EB, EB)
    d1 = edge_index_counter[1].reshape(E // EB, EB)
    s2 = edge_index_support[0].reshape(E // EB, EB)
    d2 = edge_index_support[1].reshape(E // EB, EB)

    acc = _sc_edges(whp_t, whp_p, whp_p2, t_t, t_p, t_p2, t_c,
                    s0, d0, s1, d1, s2, d2)

    tv = jnp.asarray(t)
    ccf = jnp.where(tv > 1, CC, 1.0).astype(f32)
    csf = jnp.where(tv > 1, 1.0 - CC, 0.0).astype(f32)
    g = jnp.where(tv > 0, 1.0, 0.0).astype(f32)
    coef = jnp.stack([ccf, csf, g]).reshape(1, 3)

    return _finish(acc, weight_ih, weight_hh,
                   bias_ih.reshape(1, -1), bias_hh.reshape(1, -1), coef)
